# baseline (speedup 1.0000x reference)
import jax
import jax.numpy as jnp
from jax import lax
from jax.experimental import pallas as pl
from jax.experimental.pallas import tpu as pltpu
from jax.experimental.pallas import tpu_sc as plsc

_NX = 128
_C = 64
_LAMDIFF = 0.01
_DELTA = 0.001
_N = 320000

_NC = 2
_NS = 16
_NW = _NC * _NS
_G = 128
_NGRP = _N // _G

_B = 1024
_NTC = 102400
_NB = _NTC // _B
_G0 = _NTC // _G
_ROUNDS = -(-(_NGRP - _G0) // _NW)


def _sc_segment_sums(x_hbm, y_hbm, zsum_hbm,
                     sums_out,
                     xbuf0, xbuf1, ybuf0, ybuf1, sem0, sem1,
                     ssum):
    c = lax.axis_index("c")
    s = lax.axis_index("s")
    wid = c * _NS + s
    xbufs = (xbuf0, xbuf1)
    ybufs = (ybuf0, ybuf1)
    sems = (sem0, sem1)
    @pl.when(s == 0)
    def _():
        pltpu.sync_copy(zsum_hbm, ssum)

    plsc.subcore_barrier()

    def start_load(g, b):
        @pl.when(g < _NGRP)
        def _():
            pltpu.async_copy(x_hbm.at[pl.ds(g * _G, _G)], xbufs[b], sems[b])
            pltpu.async_copy(y_hbm.at[g], ybufs[b], sems[b])

    def wait_load(g, b):
        pltpu.make_async_copy(x_hbm.at[pl.ds(g * _G, _G)], xbufs[b],
                              sems[b]).wait()
        pltpu.make_async_copy(y_hbm.at[g], ybufs[b], sems[b]).wait()

    def consume(g, b):
        @pl.when(g < _NGRP)
        def _():
            wait_load(g, b)
            pltpu.sync_copy(xbufs[b], ssum.at[ybufs[b]], add=True)

    start_load(_G0 + wid, 0)
    start_load(_G0 + wid + _NW, 1)

    def round_body(k, carry):
        g0 = _G0 + wid + (2 * k) * _NW
        g1 = _G0 + wid + (2 * k + 1) * _NW
        consume(g0, 0)
        start_load(g0 + 2 * _NW, 0)
        consume(g1, 1)
        start_load(g1 + 2 * _NW, 1)
        return carry

    lax.fori_loop(0, -(-_ROUNDS // 2), round_body, 0)
    plsc.subcore_barrier()

    @pl.when(s == 0)
    def _():
        pltpu.sync_copy(ssum, sums_out.at[c])


@jax.jit
def _sc_call(x, y2, zsum):
    mesh = plsc.VectorSubcoreMesh(core_axis_name="c", subcore_axis_name="s",
                                  num_cores=_NC, num_subcores=_NS)
    return pl.kernel(
        _sc_segment_sums,
        out_type=jax.ShapeDtypeStruct((_NC, _C, _NX), jnp.float32),
        mesh=mesh,
        scratch_types=[
            pltpu.VMEM((_G, _NX), jnp.float32),
            pltpu.VMEM((_G, _NX), jnp.float32),
            pltpu.VMEM((_G,), jnp.int32),
            pltpu.VMEM((_G,), jnp.int32),
            pltpu.SemaphoreType.DMA,
            pltpu.SemaphoreType.DMA,
            pltpu.VMEM_SHARED((_C, _NX), jnp.float32),
        ],
    )(x, y2, zsum)


def _tc_segsum(y_ref, yall_ref, x_ref, sums_ref, cnts_ref):
    i = pl.program_id(0)
    labels = y_ref[0, 0, :]
    oh = (labels[:, None]
          == lax.broadcasted_iota(jnp.int32, (1, _C), 1)).astype(jnp.float32)
    part = lax.dot_general(oh, x_ref[...], (((0,), (0,)), ((), ())),
                           preferred_element_type=jnp.float32)
    la = yall_ref[0, 0, :]
    cnt = jnp.sum((la[:, None]
                   == lax.broadcasted_iota(jnp.int32, (1, _C), 1)
                   ).astype(jnp.float32), axis=0)[:, None]

    @pl.when(i == 0)
    def _():
        sums_ref[...] = part
        cnts_ref[...] = cnt

    @pl.when(i > 0)
    def _():
        sums_ref[...] += part
        cnts_ref[...] += cnt


def _tc_epilogue(sums_ref, tsum_ref, tcnt_ref,
                 m1_ref, m2_ref, l1_ref, l2_ref, o_ref):
    sums = sums_ref[0] + sums_ref[1] + tsum_ref[...]
    counts = tcnt_ref[...]
    xbar = sums / counts
    m1 = m1_ref[...]
    m2 = m2_ref[...]
    dd1 = xbar - m1
    dd2 = xbar - m2
    d1 = jnp.sum(dd1 * dd1, axis=1, keepdims=True)
    d2 = jnp.sum(dd2 * dd2, axis=1, keepdims=True)
    cond = d1 < d2
    l1 = l1_ref[...]
    l2 = l2_ref[...]
    l1a = jnp.clip(l1 - _DELTA, 0.0, 1.0)
    l2a = l1a + _LAMDIFF
    l2b = jnp.clip(l2 + _DELTA, 0.0, 1.0)
    l1b = l2b - _LAMDIFF
    l1n = jnp.where(cond, l1a, l1b)
    l2n = jnp.where(cond, l2a, l2b)
    m1n = (1.0 - l1n) * xbar + l1n * m1
    m2n = (1.0 - l2n) * xbar + l2n * m2
    me = 0.5 * (m1n + m2n)
    colsum = jnp.sum(me, axis=0, keepdims=True)
    val = _C * jnp.sum(me * me) - jnp.sum(colsum * colsum)
    o_ref[...] = jnp.sqrt(jnp.maximum(val, 0.0)).reshape(1, 1)


def kernel(x, y, m1, m2, lam1, lam2):
    y2 = y.reshape(_NGRP, _G)
    zsum = jnp.zeros((_C, _NX), jnp.float32)
    sums = _sc_call(x, y2, zsum)

    y_tc = y[:_NTC].reshape(_NB, 1, _B)
    tsum, tcnt = pl.pallas_call(
        _tc_segsum,
        grid=(_NB,),
        in_specs=[
            pl.BlockSpec((1, 1, _B), lambda i: (i, 0, 0)),
            pl.BlockSpec((1, 1, _N // _NB), lambda i: (i, 0, 0)),
            pl.BlockSpec((_B, _NX), lambda i: (i, 0)),
        ],
        out_specs=[
            pl.BlockSpec((_C, _NX), lambda i: (0, 0)),
            pl.BlockSpec((_C, 1), lambda i: (0, 0)),
        ],
        out_shape=[
            jax.ShapeDtypeStruct((_C, _NX), jnp.float32),
            jax.ShapeDtypeStruct((_C, 1), jnp.float32),
        ],
    )(y_tc, y.reshape(_NB, 1, _N // _NB), x)

    out = pl.pallas_call(
        _tc_epilogue,
        out_shape=jax.ShapeDtypeStruct((1, 1), jnp.float32),
    )(sums, tsum, tcnt, m1, m2, lam1.reshape(_C, 1), lam2.reshape(_C, 1))
    return out[0, 0]
